# baseline (speedup 1.0000x reference)
import jax
import jax.numpy as jnp
from jax import lax
from jax.experimental import pallas as pl
from jax.experimental.pallas import tpu as pltpu
from jax.experimental.pallas import tpu_sc as plsc

N_USERS = 4096
N_ITEMS = 1000
H = 12
A = 1 << H
R = 256
NW = 32
UPW = N_USERS // NW
G = 16
NG = UPW // G
WORDS = N_ITEMS // 4


def _eb_body(dist_ref, ebt_ref):
    dist = dist_ref[...]
    k_ids = lax.broadcasted_iota(jnp.int32, (A, 128), 0)
    j_ids = jnp.minimum(lax.broadcasted_iota(jnp.int32, (A, 128), 1), 31)
    bitmat = ((k_ids >> j_ids) & 1).astype(jnp.float32)
    ebT = lax.dot_general(bitmat, dist, (((0,), (1,)), ((), ())),
                          preferred_element_type=jnp.float32)
    ebt_ref[...] = ebT[:16, :]


def _sc_body(ebt_hbm, hist_hbm, o_hbm, act_hbm, num_hbm,
             eb_v, hist_v, obuf, wbuf, num_v):
    wid = lax.axis_index("s") * 2 + lax.axis_index("c")
    base = wid * UPW
    pltpu.sync_copy(ebt_hbm.at[:, pl.ds(base, UPW)], eb_v)
    pltpu.sync_copy(hist_hbm.at[:, pl.ds(base, UPW)], hist_v)

    zf = jnp.zeros((G,), jnp.float32)
    zi = jnp.zeros((G,), jnp.int32)

    def zero_o(i, carry):
        obuf[pl.ds(i * 16, 16)] = zf
        return carry

    lax.fori_loop(0, G * N_ITEMS // 16, zero_o, 0)

    def zero_w(i, carry):
        wbuf[pl.ds(i * 16, 16)] = zi
        return carry

    lax.fori_loop(0, G * WORDS // 16, zero_w, 0)

    rows = lax.broadcasted_iota(jnp.int32, (G,), 0)
    row_o = rows * N_ITEMS
    row_w = rows * WORDS

    def group(g, carry):
        cbase = g * G
        for j in range(H):
            col = hist_v[j, pl.ds(cbase, G)]
            val = eb_v[j, pl.ds(cbase, G)]
            plsc.addupdate_scatter(obuf, [row_o + col], val)
        for j in range(H):
            col = hist_v[j, pl.ds(cbase, G)]
            oval = plsc.load_gather(obuf, [row_o + col])
            bit = (oval > 0.5).astype(jnp.int32)
            widx = row_w + (col >> 2)
            sh = (col & 3) * 8
            wold = plsc.load_gather(wbuf, [widx])
            wnew = (wold & jnp.bitwise_not(jnp.left_shift(jnp.int32(255), sh))) \
                | (bit << sh)
            plsc.store_scatter(wbuf, [widx], wnew)
        num = jnp.zeros((G,), jnp.int32)
        for c in range(H):
            oval = plsc.load_gather(obuf, [row_o + c])
            num = num | ((oval > 0.5).astype(jnp.int32) << c)
        num_v[pl.ds(cbase, G)] = num
        u0 = base + cbase
        pltpu.sync_copy(obuf, o_hbm.at[pl.ds(u0 * N_ITEMS, G * N_ITEMS)])
        pltpu.sync_copy(wbuf, act_hbm.at[pl.ds(u0 * WORDS, G * WORDS)])
        for j in range(H):
            col = hist_v[j, pl.ds(cbase, G)]
            plsc.store_scatter(obuf, [row_o + col], zf)
            plsc.store_scatter(wbuf, [row_w + (col >> 2)], zi)
        return carry

    lax.fori_loop(0, NG, group, 0)
    pltpu.sync_copy(num_v, num_hbm.at[pl.ds(base, UPW)])


def _make_sc_call(interpret=False):
    mesh = plsc.VectorSubcoreMesh(
        core_axis_name="c", subcore_axis_name="s", num_cores=2, num_subcores=16
    )
    return pl.kernel(
        _sc_body,
        out_type=[
            jax.ShapeDtypeStruct((N_USERS * N_ITEMS,), jnp.float32),
            jax.ShapeDtypeStruct((N_USERS * WORDS,), jnp.int32),
            jax.ShapeDtypeStruct((N_USERS,), jnp.int32),
        ],
        mesh=mesh,
        scratch_types=[
            pltpu.VMEM((16, UPW), jnp.float32),
            pltpu.VMEM((16, UPW), jnp.int32),
            pltpu.VMEM((G * N_ITEMS,), jnp.float32),
            pltpu.VMEM((G * WORDS,), jnp.int32),
            pltpu.VMEM((UPW,), jnp.int32),
        ],
        compiler_params=pltpu.CompilerParams(needs_layout_passes=False),
        interpret=interpret,
    )


def kernel(distribution, history):
    hist = history.astype(jnp.int32)
    hist_t = jnp.zeros((16, N_USERS), jnp.int32).at[:H].set(hist.T)
    ebt = pl.pallas_call(
        _eb_body,
        grid=(N_USERS // R,),
        in_specs=[pl.BlockSpec((R, A), lambda i: (i, 0))],
        out_specs=pl.BlockSpec((16, R), lambda i: (0, i)),
        out_shape=jax.ShapeDtypeStruct((16, N_USERS), jnp.float32),
    )(distribution)
    o_flat, act_words, num = _make_sc_call()(ebt, hist_t)
    o = o_flat.reshape(N_USERS, N_ITEMS)
    act = (
        lax.bitcast_convert_type(act_words, jnp.uint8)
        .reshape(N_USERS, N_ITEMS)
        .astype(jnp.bool_)
    )
    return (o, act, num)
